# Initial kernel scaffold; baseline (speedup 1.0000x reference)
import jax
import jax.numpy as jnp
from jax.experimental import pallas as pl


def kernel(lhs, rhs, params):
    raise NotImplementedError("write your pallas kernel here")


# speedup vs baseline: 43.4617x; 43.4617x over previous
import numpy as np
import jax
import jax.numpy as jnp
from jax.experimental import pallas as pl
from jax.experimental.pallas import tpu as pltpu

_SELU_SCALE = 1.0507009873554805
_SELU_ALPHA = 1.6732632423543772
_BN_SQRT = np.float32(np.sqrt(np.float32(1.0 + 1e-5)))

_W_NAMES = ('pt1', 'pt2', 'att', 'attM', 'wa', 'woa', 'waM', 'woaM')
_M_NAMES = ('m11', 'm12', 'm22', 'mM')


def _dot_bf(a, b):
    return jax.lax.dot_general(a, b, (((1,), (1,)), ((), ())),
                               preferred_element_type=jnp.float32)


def _mat_hi(a, b):
    return jax.lax.dot_general(a, b, (((1,), (0,)), ((), ())),
                               preferred_element_type=jnp.float32,
                               precision=jax.lax.Precision.HIGHEST)


def _row(col):
    return jax.lax.dot_general(jnp.ones((1, 1), jnp.float32), col,
                               (((1,), (1,)), ((), ())),
                               preferred_element_type=jnp.float32,
                               precision=jax.lax.Precision.HIGHEST)


def _selu(z):
    zn = jnp.minimum(z, 0.0)
    return _SELU_SCALE * jnp.where(z > 0, z, _SELU_ALPHA * (jnp.exp(zn) - 1.0))


def _att_group(xi, x_all, attw, attb, regions, n_valid):
    g, d = xi.shape
    np_ = x_all.shape[0]
    scaled = (x_all[:, None, :] * xi[None, :, :]).reshape(np_ * g, d)
    am = jnp.tanh(_dot_bf(scaled, attw) + attb)
    jrow = jax.lax.broadcasted_iota(jnp.int32, (np_, g, d), 0)
    alpha = jnp.zeros((np_, g, d), jnp.float32)
    cache = {}
    for lo, hi, m_rep in regions:
        key = id(m_rep)
        if key not in cache:
            cache[key] = _dot_bf(am, m_rep).reshape(np_, g, d)
        alpha = jnp.where((jrow >= lo) & (jrow < hi), cache[key], alpha)
    mask = jrow < n_valid
    mseg = jnp.max(jnp.where(mask, alpha, -jnp.inf), axis=0)
    e = jnp.exp(alpha - mseg[None]) * mask.astype(jnp.float32)
    s = jnp.zeros((g, d), jnp.float32)
    for j in range(n_valid):
        s = s + e[j]
    a = e / (s[None] + 1e-16)
    w = jnp.zeros((g, d), jnp.float32)
    for j in range(n_valid):
        w = w + a[j] * x_all[j:j + 1, :]
    return w


def _gal(dl, dr, m, w, b, mv):
    nl, nr = dl.shape[0], dr.shape[0]
    d = dl.shape[1]
    n = nl + nr + 1
    np_ = ((n + 7) // 8) * 8
    xl = _dot_bf(dl, w['pt1']) + b['pt1']
    xr = _dot_bf(dr, w['pt2']) + b['pt2']
    x_all = jnp.concatenate(
        [xl, xr, m, jnp.zeros((np_ - n, d), jnp.float32)], axis=0)
    wl = _att_group(xl, x_all, w['att'], b['att'],
                    [(0, nl, mv['m11']), (nl, nl + nr, mv['m12'])], n)
    wr = _att_group(xr, x_all, w['att'], b['att'],
                    [(0, nl, mv['m12']), (nl, nl + nr, mv['m22'])], n)
    wm = _att_group(m, x_all, w['attM'], b['attM'],
                    [(0, n, mv['mM'])], n)
    out_l = (_dot_bf(wl, w['wa']) + b['wa']) + (_dot_bf(xl, w['woa']) + b['woa'])
    out_r = (_dot_bf(wr, w['wa']) + b['wa']) + (_dot_bf(xr, w['woa']) + b['woa'])
    out_m = (_dot_bf(wm, w['waM']) + b['waM']) + (_dot_bf(m, w['woaM']) + b['woaM'])
    return (_selu(out_l / _BN_SQRT), _selu(out_r / _BN_SQRT), out_m)


def _pool(x, pw_rep, pb_rep, k):
    g, d = x.shape
    ymat = jax.nn.sigmoid(_dot_bf(x, pw_rep) + pb_rep)
    yrow = _row(ymat[:, :1])
    yrowmat = jnp.broadcast_to(yrow, (g, g))
    yimat = ymat[:, :g]
    jrow = jax.lax.broadcasted_iota(jnp.int32, (g, g), 1)
    icol = jax.lax.broadcasted_iota(jnp.int32, (g, g), 0)
    beats = (yrowmat > yimat) | ((yrowmat == yimat) & (jrow < icol))
    rank = jnp.sum(beats.astype(jnp.float32), axis=1, keepdims=True)
    rankrow = _row(rank)
    rid = jax.lax.broadcasted_iota(jnp.int32, (k, g), 0).astype(jnp.float32)
    perm = (rid == jnp.broadcast_to(rankrow, (k, g))).astype(jnp.float32)
    return _mat_hi(perm, x * ymat)


def _body(lhs_ref, rhs_ref, ws_ref, bs_ref, ms_ref, pw_ref, pb_ref,
          mst_ref, hw_ref, hb_ref, out_ref):
    xl0 = lhs_ref[0]
    xr0 = rhs_ref[0]
    res = []
    for bi in range(2):
        dl, dr = xl0, xr0
        m = mst_ref[bi]
        for li in range(2):
            w = {nm: ws_ref[bi, li, wi] for wi, nm in enumerate(_W_NAMES)}
            b = {nm: bs_ref[bi, li, wi] for wi, nm in enumerate(_W_NAMES)}
            mv = {nm: ms_ref[bi, li, mi] for mi, nm in enumerate(_M_NAMES)}
            act_l, act_r, m = _gal(dl, dr, m, w, b, mv)
            k = act_l.shape[0] // 2
            dl = _pool(act_l, pw_ref[bi, li, 0], pb_ref[bi, li, 0], k)
            dr = _pool(act_r, pw_ref[bi, li, 1], pb_ref[bi, li, 1], k)
        res.append((dl, dr, m))
    big_l = jnp.maximum(res[0][0], res[1][0])
    big_r = jnp.maximum(res[0][1], res[1][1])
    big_m = jnp.maximum(res[0][2], res[1][2])
    hidden = jnp.concatenate([
        jnp.max(big_l, axis=0, keepdims=True),
        jnp.max(big_r, axis=0, keepdims=True),
        jnp.mean(big_l, axis=0, keepdims=True),
        jnp.mean(big_r, axis=0, keepdims=True),
        big_m,
    ], axis=1)
    out_ref[0] = _dot_bf(hidden, hw_ref[...]) + hb_ref[...]


def kernel(lhs, rhs, params):
    bs, nl, d = lhs.shape
    branches = ('first', 'second')

    ws = jnp.stack([jnp.stack([jnp.stack(
        [params[br]['gal'][li][nm + '_w'] for nm in _W_NAMES])
        for li in range(2)]) for br in branches])
    bs_ = jnp.stack([jnp.stack([jnp.stack(
        [params[br]['gal'][li][nm + '_b'].reshape(1, d) for nm in _W_NAMES])
        for li in range(2)]) for br in branches])
    ms = jnp.stack([jnp.stack([jnp.stack(
        [jnp.broadcast_to(params[br]['gal'][li][nm], (d, d))
         for nm in _M_NAMES])
        for li in range(2)]) for br in branches])
    pw = jnp.stack([jnp.stack([jnp.stack(
        [jnp.broadcast_to(params[br]['pool'][li][si]['w'], (d, d))
         for si in range(2)])
        for li in range(2)]) for br in branches])
    pb = jnp.stack([jnp.stack([jnp.stack(
        [jnp.broadcast_to(params[br]['pool'][li][si]['b'].reshape(1, 1),
                          (1, d)) for si in range(2)])
        for li in range(2)]) for br in branches])
    mst = jnp.stack([params[br]['master'].reshape(1, d) for br in branches])
    hw = jnp.zeros((128, 5 * d), jnp.float32).at[:2].set(params['head_w'])
    hb = jnp.zeros((1, 128), jnp.float32).at[0, :2].set(params['head_b'])

    full = lambda a: pl.BlockSpec(a.shape, lambda i: (0,) * a.ndim)
    out = pl.pallas_call(
        _body,
        grid=(bs,),
        in_specs=[
            pl.BlockSpec((1, nl, d), lambda i: (i, 0, 0)),
            pl.BlockSpec((1, rhs.shape[1], d), lambda i: (i, 0, 0)),
            full(ws), full(bs_), full(ms), full(pw), full(pb),
            full(mst), full(hw), full(hb),
        ],
        out_specs=pl.BlockSpec((1, 1, 128), lambda i: (i, 0, 0)),
        out_shape=jax.ShapeDtypeStruct((bs, 1, 128), jnp.float32),
        compiler_params=pltpu.CompilerParams(
            dimension_semantics=("parallel",)),
    )(lhs, rhs, ws, bs_, ms, pw, pb, mst, hw, hb)
    return out[:, 0, :2]
